# Initial kernel scaffold; baseline (speedup 1.0000x reference)
import jax
import jax.numpy as jnp
from jax.experimental import pallas as pl


def kernel(x, router_w, gate_w, up_w, down_w):
    raise NotImplementedError("write your pallas kernel here")


# speedup vs baseline: 2.6308x; 2.6308x over previous
import functools

import jax
import jax.numpy as jnp
from jax import lax
from jax.experimental import pallas as pl
from jax.experimental.pallas import tpu as pltpu
from jax.experimental.pallas import tpu_sc as plsc


def _router_body(x_ref, rw_ref, dst0_ref, dst1_ref, loc0_ref, loc1_ref,
                 w0_ref, w1_ref, base_ref, fb_ref, *, C, E, TB):
    b = pl.program_id(0)

    @pl.when(b == 0)
    def _():
        base_ref[...] = jnp.zeros_like(base_ref)

    xb = x_ref[...]
    rw = rw_ref[...]
    logits = lax.dot_general(xb, rw, (((1,), (1,)), ((), ())),
                             preferred_element_type=jnp.float32)
    scores = jax.nn.sigmoid(logits)

    iota_e = lax.broadcasted_iota(jnp.int32, (TB, E), 1).astype(jnp.float32)
    m1 = jnp.max(scores, axis=1, keepdims=True)
    i1 = jnp.min(jnp.where(scores == m1, iota_e, float(E)), axis=1, keepdims=True)
    masked = jnp.where(iota_e == i1, -1.0, scores)
    m2 = jnp.max(masked, axis=1, keepdims=True)
    i2 = jnp.min(jnp.where(masked == m2, iota_e, float(E)), axis=1, keepdims=True)
    denom = jnp.maximum(m1 + m2, 1e-9)
    w1 = m1 / denom
    w2 = m2 / denom

    oh1 = (iota_e == i1).astype(jnp.float32)
    oh2 = (iota_e == i2).astype(jnp.float32)
    cnt = oh1 + oh2
    r_iota = lax.broadcasted_iota(jnp.int32, (TB, TB), 0)
    c_iota = lax.broadcasted_iota(jnp.int32, (TB, TB), 1)
    tri = (r_iota > c_iota).astype(jnp.float32)
    prior = lax.dot_general(tri, cnt, (((1,), (0,)), ((), ())),
                            preferred_element_type=jnp.float32)
    base = base_ref[...]
    pos1 = jnp.sum(oh1 * (prior + base), axis=1, keepdims=True)
    pos2 = jnp.sum(oh2 * (prior + base), axis=1, keepdims=True)
    base_ref[...] = base + jnp.sum(cnt, axis=0, keepdims=True)

    keep1 = pos1 < C
    keep2 = pos2 < C
    slot1 = i1 * C + pos1
    slot2 = i2 * C + pos2

    @pl.when(b == 0)
    def _():
        r0 = lax.broadcasted_iota(jnp.int32, (TB, 1), 0)
        fbv = jnp.sum(jnp.where(r0 == 0, slot1, 0.0))
        fb_ref[0, 0] = fbv.astype(jnp.int32)

    fb = fb_ref[0, 0].astype(jnp.float32)
    trash = float(E * C)
    dst0_ref[...] = jnp.where(keep1, slot1, trash).astype(jnp.int32)
    dst1_ref[...] = jnp.where(keep2, slot2, trash).astype(jnp.int32)
    loc0_ref[...] = jnp.where(keep1, slot1, fb).astype(jnp.int32)
    loc1_ref[...] = jnp.where(keep2, slot2, fb).astype(jnp.int32)
    w0_ref[...] = jnp.broadcast_to(jnp.where(keep1, w1, 0.0), (TB, 16))
    w1_ref[...] = jnp.broadcast_to(jnp.where(keep2, w2, 0.0), (TB, 16))


def _router_meta(x, router_w, *, C, TB=256):
    T, D = x.shape
    E = router_w.shape[0]
    grid = (T // TB,)
    body = functools.partial(_router_body, C=C, E=E, TB=TB)
    return pl.pallas_call(
        body,
        grid=grid,
        in_specs=[
            pl.BlockSpec((TB, D), lambda b: (b, 0)),
            pl.BlockSpec((E, D), lambda b: (0, 0)),
        ],
        out_specs=[
            pl.BlockSpec((TB, 1), lambda b: (b, 0)),
            pl.BlockSpec((TB, 1), lambda b: (b, 0)),
            pl.BlockSpec((TB, 1), lambda b: (b, 0)),
            pl.BlockSpec((TB, 1), lambda b: (b, 0)),
            pl.BlockSpec((TB, 16), lambda b: (b, 0)),
            pl.BlockSpec((TB, 16), lambda b: (b, 0)),
        ],
        out_shape=[
            jax.ShapeDtypeStruct((T, 1), jnp.int32),
            jax.ShapeDtypeStruct((T, 1), jnp.int32),
            jax.ShapeDtypeStruct((T, 1), jnp.int32),
            jax.ShapeDtypeStruct((T, 1), jnp.int32),
            jax.ShapeDtypeStruct((T, 16), jnp.float32),
            jax.ShapeDtypeStruct((T, 16), jnp.float32),
        ],
        scratch_shapes=[
            pltpu.VMEM((1, E), jnp.float32),
            pltpu.SMEM((1, 1), jnp.int32),
        ],
    )(x, router_w)


def _mlp_body(ein_ref, gw_ref, uw_ref, dw_ref, y_ref):
    xin = ein_ref[...]
    g = lax.dot_general(xin, gw_ref[0], (((1,), (1,)), ((), ())),
                        preferred_element_type=jnp.float32)
    u = lax.dot_general(xin, uw_ref[0], (((1,), (1,)), ((), ())),
                        preferred_element_type=jnp.float32)
    act = g * jax.nn.sigmoid(g) * u
    y_ref[...] = lax.dot_general(act, dw_ref[0], (((1,), (1,)), ((), ())),
                                 preferred_element_type=jnp.float32)


def _mlp(ein, gate_w, up_w, down_w, *, C):
    E, F, D = gate_w.shape
    return pl.pallas_call(
        _mlp_body,
        grid=(E,),
        in_specs=[
            pl.BlockSpec((C, D), lambda e: (e, 0)),
            pl.BlockSpec((1, F, D), lambda e: (e, 0, 0)),
            pl.BlockSpec((1, F, D), lambda e: (e, 0, 0)),
            pl.BlockSpec((1, D, F), lambda e: (e, 0, 0)),
        ],
        out_specs=pl.BlockSpec((C, D), lambda e: (e, 0)),
        out_shape=jax.ShapeDtypeStruct((E * C, D), jnp.float32),
    )(ein, gate_w, up_w, down_w)


def _dispatch(x, dst0, dst1, *, C, E):
    T, D = x.shape
    info = plsc.get_sparse_core_info()
    NC, NS = info.num_cores, info.num_subcores
    NW = NC * NS
    tok_per_w = T // NW
    CH = 64

    mesh = plsc.VectorSubcoreMesh(core_axis_name="c", subcore_axis_name="s")

    @functools.partial(
        pl.kernel,
        out_type=jax.ShapeDtypeStruct((E * C + 8, D), jnp.float32),
        mesh=mesh,
        scratch_types=[
            pltpu.VMEM((CH, D), jnp.float32),
            pltpu.VMEM((2, CH), jnp.int32),
            pltpu.SemaphoreType.DMA,
        ],
    )
    def disp(x_hbm, dst0_hbm, dst1_hbm, ein_hbm, rows_v, idx_v, sem):
        wid = lax.axis_index("s") * NC + lax.axis_index("c")
        base = wid * tok_per_w
        for c in range(tok_per_w // CH):
            tb = base + c * CH
            pltpu.sync_copy(dst0_hbm.at[pl.ds(tb, CH)], idx_v.at[0])
            pltpu.sync_copy(dst1_hbm.at[pl.ds(tb, CH)], idx_v.at[1])
            pltpu.sync_copy(x_hbm.at[pl.ds(tb, CH)], rows_v)
            cp0 = pltpu.async_copy(rows_v, ein_hbm.at[idx_v.at[0]], sem)
            cp1 = pltpu.async_copy(rows_v, ein_hbm.at[idx_v.at[1]], sem)
            cp0.wait()
            cp1.wait()

    return disp(x, dst0, dst1)


def _combine(y, loc0, loc1, w0b, w1b, *, T, D):
    info = plsc.get_sparse_core_info()
    NC, NS = info.num_cores, info.num_subcores
    NW = NC * NS
    tok_per_w = T // NW
    TOKC = 32
    L = 16

    mesh = plsc.VectorSubcoreMesh(core_axis_name="c", subcore_axis_name="s")

    @functools.partial(
        pl.kernel,
        out_type=jax.ShapeDtypeStruct((T, D), jnp.float32),
        mesh=mesh,
        scratch_types=[
            pltpu.VMEM((TOKC, D), jnp.float32),
            pltpu.VMEM((TOKC, D), jnp.float32),
            pltpu.VMEM((TOKC, D), jnp.float32),
            pltpu.VMEM((TOKC, L), jnp.float32),
            pltpu.VMEM((TOKC, L), jnp.float32),
            pltpu.VMEM((TOKC,), jnp.int32),
            pltpu.VMEM((TOKC,), jnp.int32),
            pltpu.SemaphoreType.DMA,
        ],
    )
    def comb(y_hbm, loc0_hbm, loc1_hbm, w0_hbm, w1_hbm, out_hbm,
             r0_v, r1_v, o_v, w0_v, w1_v, l0_v, l1_v, sem):
        wid = lax.axis_index("s") * NC + lax.axis_index("c")
        base = wid * tok_per_w
        for c in range(tok_per_w // TOKC):
            tb = base + c * TOKC
            pltpu.sync_copy(loc0_hbm.at[pl.ds(tb, TOKC)], l0_v)
            pltpu.sync_copy(loc1_hbm.at[pl.ds(tb, TOKC)], l1_v)
            pltpu.sync_copy(w0_hbm.at[pl.ds(tb, TOKC)], w0_v)
            pltpu.sync_copy(w1_hbm.at[pl.ds(tb, TOKC)], w1_v)
            cp0 = pltpu.async_copy(y_hbm.at[l0_v], r0_v, sem)
            cp1 = pltpu.async_copy(y_hbm.at[l1_v], r1_v, sem)
            cp0.wait()
            cp1.wait()

            def tok_body(j, carry):
                w0 = w0_v[j]
                w1 = w1_v[j]
                for l in range(D // L):
                    sl = pl.ds(l * L, L)
                    o_v[j, sl] = w0 * r0_v[j, sl] + w1 * r1_v[j, sl]
                return carry

            lax.fori_loop(0, TOKC, tok_body, 0)
            pltpu.sync_copy(o_v, out_hbm.at[pl.ds(tb, TOKC)])

    return comb(y, loc0, loc1, w0b, w1b)


def kernel(x, router_w, gate_w, up_w, down_w):
    T, D = x.shape
    E = router_w.shape[0]
    K = 2
    C = int(2.0 * T * K / E)

    dst0, dst1, loc0, loc1, w0b, w1b = _router_meta(x, router_w, C=C)
    ein = _dispatch(x, dst0.reshape(T), dst1.reshape(T), C=C, E=E)
    y = _mlp(ein, gate_w, up_w, down_w, C=C)
    out = _combine(y, loc0.reshape(T), loc1.reshape(T), w0b, w1b, T=T, D=D)
    return out
